# baseline (speedup 1.0000x reference)
import jax
import jax.numpy as jnp
from jax import lax
from jax.experimental import pallas as pl
from jax.experimental.pallas import tpu as pltpu
from jax.experimental.pallas import tpu_sc as plsc

B = 4096
D = 32
VIS = 128
PRV = 9
NC = 2
NS = 16
NW = NC * NS
BPW = B // NW

L_DE, L_SK, L_HE = 50, 8, 5
T_DE, T_SK, T_HE = BPW * L_DE, BPW * L_SK, BPW * L_HE

IT_DE, IT_SK, IT_HE = 16, 64, 128
NCH_DE, NCH_SK = BPW // IT_DE, BPW // IT_SK
ROWS_MAX = max(IT_DE * L_DE, IT_SK * L_SK, IT_HE * L_HE)
NBUF = 3


def _sum_chunk(buf, sums, col0, item_base, n_items, ln):
  nacc = min(4, ln)

  @pl.loop(0, n_items)
  def _(i):
    r0 = i * ln
    acc = [[buf[r0 + k, pl.ds(16 * h, 16)] for k in range(nacc)]
           for h in range(2)]
    for t in range(nacc, ln):
      acc[0][t % nacc] = acc[0][t % nacc] + buf[r0 + t, pl.ds(0, 16)]
      acc[1][t % nacc] = acc[1][t % nacc] + buf[r0 + t, pl.ds(16, 16)]
    row = item_base + i
    for h in range(2):
      tot = acc[h][0]
      for k in range(1, nacc):
        tot = tot + acc[h][k]
      sums[row, pl.ds(col0 + 16 * h, 16)] = tot


def _run_chunks(chunks, bufs, gsems):
  n = len(chunks)

  def fire(c):
    tab, idx_v, off, rows, _, _, _, _, _ = chunks[c]
    b = c % NBUF
    return pltpu.async_copy(tab.at[idx_v.at[pl.ds(off, rows)]],
                            bufs[b].at[pl.ds(0, rows)], gsems[b])

  g = {}
  for c in range(min(NBUF, n)):
    g[c] = fire(c)
  for c in range(n):
    g[c].wait()
    _, _, _, _, sums, col0, item_base, n_items, ln = chunks[c]
    _sum_chunk(bufs[c % NBUF], sums, col0, item_base, n_items, ln)
    if c + NBUF < n:
      g[c + NBUF] = fire(c + NBUF)


def _sc_pool_body(de_tab, sk_tab, he_tab, de_idx, sk_idx, he_idx,
                  out, idx_de_v, idx_sk_v, idx_he_v, b0, b1, b2,
                  sums, s0, s1, s2, osem):
  cid = lax.axis_index("c")
  sid = lax.axis_index("s")
  base = (sid * NC + cid) * BPW

  with jax.named_scope("idx_stage"):
    st = [
        pltpu.async_copy(de_idx.at[pl.ds(base * L_DE, T_DE)], idx_de_v, osem),
        pltpu.async_copy(sk_idx.at[pl.ds(base * L_SK, T_SK)], idx_sk_v, osem),
        pltpu.async_copy(he_idx.at[pl.ds(base * L_HE, T_HE)], idx_he_v, osem),
    ]
    for c in st:
      c.wait()

  z = jnp.zeros((16,), jnp.float32)

  @pl.loop(0, BPW)
  def _(r):
    sums[r, pl.ds(96, 16)] = z
    sums[r, pl.ds(112, 16)] = z

  chunks = (
      [(de_tab, idx_de_v, c * IT_DE * L_DE, IT_DE * L_DE,
        sums, 0, c * IT_DE, IT_DE, L_DE) for c in range(NCH_DE)]
      + [(sk_tab, idx_sk_v, c * IT_SK * L_SK, IT_SK * L_SK,
          sums, 32, c * IT_SK, IT_SK, L_SK) for c in range(NCH_SK)]
      + [(he_tab, idx_he_v, 0, T_HE, sums, 64, 0, IT_HE, L_HE)]
  )
  with jax.named_scope("chunks"):
    _run_chunks(chunks, (b0, b1, b2), (s0, s1, s2))

  with jax.named_scope("out"):
    pltpu.async_copy(sums, out.at[pl.ds(base, BPW)], osem).wait()


_sc_pool = pl.kernel(
    _sc_pool_body,
    out_type=jax.ShapeDtypeStruct((B, 4 * D), jnp.float32),
    mesh=plsc.VectorSubcoreMesh(core_axis_name="c", subcore_axis_name="s",
                                num_cores=NC, num_subcores=NS),
    scratch_types=[
        pltpu.VMEM((T_DE,), jnp.int32),
        pltpu.VMEM((T_SK,), jnp.int32),
        pltpu.VMEM((T_HE,), jnp.int32),
        pltpu.VMEM((ROWS_MAX, D), jnp.float32),
        pltpu.VMEM((ROWS_MAX, D), jnp.float32),
        pltpu.VMEM((ROWS_MAX, D), jnp.float32),
        pltpu.VMEM((BPW, 4 * D), jnp.float32),
        pltpu.SemaphoreType.DMA,
        pltpu.SemaphoreType.DMA,
        pltpu.SemaphoreType.DMA,
        pltpu.SemaphoreType.DMA,
    ],
    compiler_params=pltpu.CompilerParams(use_tc_tiling_on_sc=False),
    name="sc_pool",
)


GRID = 1
TB = B // GRID


def _mlp_body(price, sums,
              d_idx, s_idx, h_idx, p_idx, vis, prtab,
              de0, sk0, he0,
              w1p, w1e, w1v, b1, w2, b2, w3, b3, out):
  s = sums[...]

  def pool(col0, idx_ref, r0_ref, ln):
    nz = jnp.sum((idx_ref[...] != 0).astype(jnp.float32), axis=1, keepdims=True)
    n0 = ln - nz
    return (s[:, col0:col0 + D] - n0 * r0_ref[...]) / jnp.maximum(nz, 1.0)

  pid = p_idx[...]
  voc = lax.broadcasted_iota(jnp.int32, (TB, PRV), 1)
  oh = jnp.where((pid == voc) & (pid != 0), 1.0, 0.0)
  pr = jnp.dot(oh, prtab[...], preferred_element_type=jnp.float32)

  de = pool(0, d_idx, de0, float(L_DE))
  sk = pool(D, s_idx, sk0, float(L_SK))
  he = pool(2 * D, h_idx, he0, float(L_HE))
  emb = jnp.concatenate([pr, de, sk, he], axis=1)

  h = (price[...] * w1p[...]
       + jnp.dot(emb, w1e[...], preferred_element_type=jnp.float32)
       + jnp.dot(vis[...], w1v[...], preferred_element_type=jnp.float32)
       + b1[...])
  h = jnp.maximum(h, 0.0)
  h = jnp.maximum(jnp.dot(h, w2[...], preferred_element_type=jnp.float32) + b2[...], 0.0)
  out[...] = jnp.dot(h, w3[...], preferred_element_type=jnp.float32) + b3[...]


def _row_spec(cols):
  return pl.BlockSpec((TB, cols), lambda i: (i, 0))


def _fix_spec(r, c):
  return pl.BlockSpec((r, c), lambda i: (0, 0))


def _vec_spec(n):
  return pl.BlockSpec((n,), lambda i: (0,))


_mlp = pl.pallas_call(
    _mlp_body,
    grid=(GRID,),
    in_specs=[
        _row_spec(1),
        _row_spec(4 * D),
        _row_spec(L_DE), _row_spec(L_SK), _row_spec(L_HE), _row_spec(1),
        _row_spec(VIS),
        _fix_spec(PRV, D),
        _fix_spec(1, D), _fix_spec(1, D), _fix_spec(1, D),
        _fix_spec(1, 256), _fix_spec(128, 256), _fix_spec(VIS, 256),
        _vec_spec(256),
        _fix_spec(256, 128), _vec_spec(128),
        _fix_spec(128, 64), _vec_spec(64),
    ],
    out_specs=pl.BlockSpec((TB, 64), lambda i: (i, 0)),
    out_shape=jax.ShapeDtypeStruct((B, 64), jnp.float32),
    name="tc_mlp",
)


def kernel(price_td, price_range_idx, description_idx, sku_idx, heir_idx,
           visual, price_range_table, desc_table, sku_table, heir_table,
           W1, b1, W2, b2, W3, b3):
  sums = _sc_pool(
      desc_table, sku_table, heir_table,
      description_idx.reshape(-1), sku_idx.reshape(-1), heir_idx.reshape(-1))

  return _mlp(price_td, sums,
              description_idx, sku_idx, heir_idx, price_range_idx, visual,
              price_range_table,
              desc_table[0:1], sku_table[0:1], heir_table[0:1],
              W1[0:1], W1[1:1 + 4 * D], W1[1 + 4 * D:], b1,
              W2, b2, W3, b3)


# speedup vs baseline: 1.0166x; 1.0166x over previous
import jax
import jax.numpy as jnp
from jax import lax
from jax.experimental import pallas as pl
from jax.experimental.pallas import tpu as pltpu
from jax.experimental.pallas import tpu_sc as plsc

B = 4096
D = 32
VIS = 128
PRV = 9
NC = 2
NS = 16
NW = NC * NS
BPW = B // NW

L_DE, L_SK, L_HE = 50, 8, 5
T_DE, T_SK, T_HE = BPW * L_DE, BPW * L_SK, BPW * L_HE

IT_DE, IT_SK, IT_HE = 16, 64, 128
NCH_DE, NCH_SK = BPW // IT_DE, BPW // IT_SK
ROWS_MAX = max(IT_DE * L_DE, IT_SK * L_SK, IT_HE * L_HE)
NBUF = 3


def _sum_chunk(buf, sums, col0, item_base, n_items, ln):
  nacc = min(4, ln)

  @pl.loop(0, n_items)
  def _(i):
    r0 = i * ln
    acc = [[buf[r0 + k, pl.ds(16 * h, 16)] for k in range(nacc)]
           for h in range(2)]
    for t in range(nacc, ln):
      acc[0][t % nacc] = acc[0][t % nacc] + buf[r0 + t, pl.ds(0, 16)]
      acc[1][t % nacc] = acc[1][t % nacc] + buf[r0 + t, pl.ds(16, 16)]
    row = item_base + i
    for h in range(2):
      tot = acc[h][0]
      for k in range(1, nacc):
        tot = tot + acc[h][k]
      sums[row, pl.ds(col0 + 16 * h, 16)] = tot


def _run_chunks(chunks, bufs, gsems):
  n = len(chunks)

  def fire(c):
    tab, idx_v, off, rows, _, _, _, _, _ = chunks[c]
    b = c % NBUF
    return pltpu.async_copy(tab.at[idx_v.at[pl.ds(off, rows)]],
                            bufs[b].at[pl.ds(0, rows)], gsems[b])

  g = {}
  for c in range(min(NBUF, n)):
    g[c] = fire(c)
  for c in range(n):
    g[c].wait()
    _, _, _, _, sums, col0, item_base, n_items, ln = chunks[c]
    _sum_chunk(bufs[c % NBUF], sums, col0, item_base, n_items, ln)
    if c + NBUF < n:
      g[c + NBUF] = fire(c + NBUF)


def _sc_desc_body(de_tab, de_idx, out, idx_de_v, b0, b1, b2,
                  sums, s0, s1, s2, osem):
  cid = lax.axis_index("c")
  sid = lax.axis_index("s")
  base = (sid * NC + cid) * BPW

  with jax.named_scope("idx_stage"):
    pltpu.sync_copy(de_idx.at[pl.ds(base * L_DE, T_DE)], idx_de_v)

  chunks = [
      (de_tab, idx_de_v, c * IT_DE * L_DE, IT_DE * L_DE,
       sums, 0, c * IT_DE, IT_DE, L_DE) for c in range(NCH_DE)
  ]
  with jax.named_scope("chunks"):
    _run_chunks(chunks, (b0, b1, b2), (s0, s1, s2))

  with jax.named_scope("out"):
    pltpu.async_copy(sums, out.at[pl.ds(base, BPW)], osem).wait()


def _sc_rest_body(sk_tab, he_tab, sk_idx, he_idx,
                  out, idx_sk_v, idx_he_v, b0, b1, b2,
                  sums, s0, s1, s2, osem):
  cid = lax.axis_index("c")
  sid = lax.axis_index("s")
  base = (sid * NC + cid) * BPW

  with jax.named_scope("idx_stage"):
    st = [
        pltpu.async_copy(sk_idx.at[pl.ds(base * L_SK, T_SK)], idx_sk_v, osem),
        pltpu.async_copy(he_idx.at[pl.ds(base * L_HE, T_HE)], idx_he_v, osem),
    ]
    for c in st:
      c.wait()

  chunks = (
      [(sk_tab, idx_sk_v, c * IT_SK * L_SK, IT_SK * L_SK,
        sums, 0, c * IT_SK, IT_SK, L_SK) for c in range(NCH_SK)]
      + [(he_tab, idx_he_v, 0, T_HE, sums, 32, 0, IT_HE, L_HE)]
  )
  with jax.named_scope("chunks"):
    _run_chunks(chunks, (b0, b1, b2), (s0, s1, s2))

  with jax.named_scope("out"):
    pltpu.async_copy(sums, out.at[pl.ds(base, BPW)], osem).wait()


_SC_MESH = plsc.VectorSubcoreMesh(core_axis_name="c", subcore_axis_name="s",
                                  num_cores=NC, num_subcores=NS)
_SC_PARAMS = pltpu.CompilerParams(use_tc_tiling_on_sc=False)


def _sc_scratch(idx_shapes):
  return ([pltpu.VMEM(s, jnp.int32) for s in idx_shapes]
          + [pltpu.VMEM((ROWS_MAX, D), jnp.float32)] * 3
          + [pltpu.VMEM((BPW, 4 * D), jnp.float32)]
          + [pltpu.SemaphoreType.DMA] * 4)


_sc_desc = pl.kernel(
    _sc_desc_body,
    out_type=jax.ShapeDtypeStruct((B, 4 * D), jnp.float32),
    mesh=_SC_MESH,
    scratch_types=_sc_scratch([(T_DE,)]),
    compiler_params=_SC_PARAMS,
    name="sc_desc",
)

_sc_rest = pl.kernel(
    _sc_rest_body,
    out_type=jax.ShapeDtypeStruct((B, 4 * D), jnp.float32),
    mesh=_SC_MESH,
    scratch_types=_sc_scratch([(T_SK,), (T_HE,)]),
    compiler_params=_SC_PARAMS,
    name="sc_rest",
)


GRID = 2
TB = B // GRID


def _mlp_body(price, sums_a, sums_b,
              d_idx, s_idx, h_idx, p_idx, vis, prtab,
              de0, sk0, he0,
              w1p, w1e, w1v, b1, w2, b2, w3, b3, out):
  sa = sums_a[...]
  sb = sums_b[...]

  def pool(s, col0, idx_ref, r0_ref, ln):
    nz = jnp.sum((idx_ref[...] != 0).astype(jnp.float32), axis=1, keepdims=True)
    n0 = ln - nz
    return (s[:, col0:col0 + D] - n0 * r0_ref[...]) / jnp.maximum(nz, 1.0)

  pid = p_idx[...]
  voc = lax.broadcasted_iota(jnp.int32, (TB, PRV), 1)
  oh = jnp.where((pid == voc) & (pid != 0), 1.0, 0.0)
  pr = jnp.dot(oh, prtab[...], preferred_element_type=jnp.float32)

  de = pool(sa, 0, d_idx, de0, float(L_DE))
  sk = pool(sb, 0, s_idx, sk0, float(L_SK))
  he = pool(sb, D, h_idx, he0, float(L_HE))
  emb = jnp.concatenate([pr, de, sk, he], axis=1)

  h = (price[...] * w1p[...]
       + jnp.dot(emb, w1e[...], preferred_element_type=jnp.float32)
       + jnp.dot(vis[...], w1v[...], preferred_element_type=jnp.float32)
       + b1[...])
  h = jnp.maximum(h, 0.0)
  h = jnp.maximum(jnp.dot(h, w2[...], preferred_element_type=jnp.float32) + b2[...], 0.0)
  out[...] = jnp.dot(h, w3[...], preferred_element_type=jnp.float32) + b3[...]


def _row_spec(cols):
  return pl.BlockSpec((TB, cols), lambda i: (i, 0))


def _fix_spec(r, c):
  return pl.BlockSpec((r, c), lambda i: (0, 0))


def _vec_spec(n):
  return pl.BlockSpec((n,), lambda i: (0,))


_mlp = pl.pallas_call(
    _mlp_body,
    grid=(GRID,),
    in_specs=[
        _row_spec(1),
        _row_spec(4 * D), _row_spec(4 * D),
        _row_spec(L_DE), _row_spec(L_SK), _row_spec(L_HE), _row_spec(1),
        _row_spec(VIS),
        _fix_spec(PRV, D),
        _fix_spec(1, D), _fix_spec(1, D), _fix_spec(1, D),
        _fix_spec(1, 256), _fix_spec(128, 256), _fix_spec(VIS, 256),
        _vec_spec(256),
        _fix_spec(256, 128), _vec_spec(128),
        _fix_spec(128, 64), _vec_spec(64),
    ],
    out_specs=pl.BlockSpec((TB, 64), lambda i: (i, 0)),
    out_shape=jax.ShapeDtypeStruct((B, 64), jnp.float32),
    name="tc_mlp",
)


def kernel(price_td, price_range_idx, description_idx, sku_idx, heir_idx,
           visual, price_range_table, desc_table, sku_table, heir_table,
           W1, b1, W2, b2, W3, b3):
  sums_a = _sc_desc(desc_table, description_idx.reshape(-1))
  sums_b = _sc_rest(sku_table, heir_table,
                    sku_idx.reshape(-1), heir_idx.reshape(-1))

  return _mlp(price_td, sums_a, sums_b,
              description_idx, sku_idx, heir_idx, price_range_idx, visual,
              price_range_table,
              desc_table[0:1], sku_table[0:1], heir_table[0:1],
              W1[0:1], W1[1:1 + 4 * D], W1[1 + 4 * D:], b1,
              W2, b2, W3, b3)
